# baseline (speedup 1.0000x reference)
import functools

import jax
import jax.numpy as jnp
from jax import lax
from jax.experimental import pallas as pl
from jax.experimental.pallas import tpu as pltpu
from jax.experimental.pallas import tpu_sc as plsc

_EPS = 1e-12
_NCH = 12
_ROW_BLK = 40
_CW = 128
_NACC = 8
_CND_BLK = 8
_SB = 8


def _sigma_body(u_ref, w_ref, s_out_ref, ssq_out_ref, acc_ref, ssq_ref):
    i = pl.program_id(0)

    @pl.when(i == 0)
    def _init():
        acc_ref[...] = jnp.zeros_like(acc_ref)
        ssq_ref[0] = 0.0

    f = w_ref.shape[1]
    nchk = f // _CW
    accs = [None] * _NACC
    for k in range(nchk):
        sl = pl.ds(k * _CW, _CW)
        c = w_ref[:, sl] * u_ref[:, sl]
        j = k % _NACC
        accs[j] = c if accs[j] is None else accs[j] + c
    while len(accs) > 1:
        accs = [a + b for a, b in zip(accs[::2], accs[1::2])]
    t1 = jnp.sum(accs[0], axis=1, keepdims=True)
    ssq_ref[0] += jnp.sum(t1 * t1)
    t1b = jnp.broadcast_to(t1, (t1.shape[0], _CW))
    for k in range(nchk):
        sl = pl.ds(k * _CW, _CW)
        acc_ref[:, sl] += t1b * w_ref[:, sl]

    @pl.when(i == pl.num_programs(0) - 1)
    def _fin():
        s_out_ref[...] = jnp.sum(acc_ref[...], axis=0, keepdims=True)
        ssq_out_ref[0, 0] = ssq_ref[0]


def _sigma_call(table, u, n_rows):
    n_cls, f = table.shape
    grid = n_rows // _ROW_BLK
    return pl.pallas_call(
        _sigma_body,
        grid=(grid,),
        in_specs=[
            pl.BlockSpec((1, f), lambda i: (0, 0)),
            pl.BlockSpec((_ROW_BLK, f), lambda i: (i, 0)),
        ],
        out_specs=[
            pl.BlockSpec((1, f), lambda i: (0, 0)),
            pl.BlockSpec(memory_space=pltpu.SMEM),
        ],
        out_shape=[
            jax.ShapeDtypeStruct((1, f), jnp.float32),
            jax.ShapeDtypeStruct((1, 1), jnp.float32),
        ],
        scratch_shapes=[
            pltpu.VMEM((_ROW_BLK, f), jnp.float32),
            pltpu.SMEM((1,), jnp.float32),
        ],
        compiler_params=pltpu.CompilerParams(
            dimension_semantics=("arbitrary",),
        ),
    )(u, table)


@functools.lru_cache(maxsize=None)
def _sc_gather_fn(n_cls, f, b):
    try:
        info = plsc.get_sparse_core_info()
        nc, ns = info.num_cores, info.num_subcores
    except Exception:
        nc, ns = 2, 16
    nw = nc * ns
    ch = f // _NCH
    ngrp = b // _SB
    nbat = ngrp * _NCH
    bpw = nbat // nw

    mesh = plsc.VectorSubcoreMesh(
        core_axis_name="c", subcore_axis_name="s",
        num_cores=nc, num_subcores=ns,
    )

    @functools.partial(
        pl.kernel,
        mesh=mesh,
        out_type=jax.ShapeDtypeStruct((b, f), jnp.float32),
        scratch_types=[
            pltpu.VMEM((_SB,), jnp.int32),
            pltpu.VMEM((_SB, ch), jnp.float32),
            pltpu.VMEM((_SB, ch), jnp.float32),
            pltpu.SemaphoreType.DMA,
            pltpu.SemaphoreType.DMA,
        ],
    )
    def gather_k(tbl_hbm, lbl_hbm, out_hbm, idx_v, b0, b1, s0, s1):
        wid = lax.axis_index("s") * nc + lax.axis_index("c")
        grp = wid % ngrp
        cb = (wid // ngrp) * bpw
        pltpu.sync_copy(lbl_hbm.at[pl.ds(grp * _SB, _SB)], idx_v)
        bufs = (b0, b1)
        sems = (s0, s1)
        copies = [
            pltpu.make_async_copy(
                tbl_hbm.at[idx_v, pl.ds((cb + k) * ch, ch)],
                bufs[k % 2],
                sems[k % 2],
            )
            for k in range(bpw)
        ]
        copies[0].start()
        for k in range(bpw):
            if k + 1 < bpw:
                copies[k + 1].start()
            copies[k].wait()
            pltpu.sync_copy(
                bufs[k % 2],
                out_hbm.at[pl.ds(grp * _SB, _SB),
                           pl.ds((cb + k) * ch, ch)])

    return gather_k


_SC_ROWS = 160
_SC_NCH = 28


@functools.lru_cache(maxsize=None)
def _sc_sigma_fn(n_cls, f, base_row):
    try:
        info = plsc.get_sparse_core_info()
        nc, ns = info.num_cores, info.num_subcores
    except Exception:
        nc, ns = 2, 16
    nw = nc * ns
    rpw = _SC_ROWS // nw
    ch = f // _SC_NCH
    nv = ch // 16

    mesh = plsc.VectorSubcoreMesh(
        core_axis_name="c", subcore_axis_name="s",
        num_cores=nc, num_subcores=ns,
    )

    wv_types = [pltpu.VMEM((1, ch), jnp.float32) for _ in range(2 * rpw)]

    @functools.partial(
        pl.kernel,
        mesh=mesh,
        out_type=[
            jax.ShapeDtypeStruct((nw, f), jnp.float32),
            jax.ShapeDtypeStruct((nw, 16), jnp.float32),
        ],
        scratch_types=wv_types + [
            pltpu.VMEM((1, f), jnp.float32),
            pltpu.VMEM((1, ch), jnp.float32),
            pltpu.VMEM((1, 16), jnp.float32),
            pltpu.SemaphoreType.DMA,
            pltpu.SemaphoreType.DMA,
        ],
    )
    def sc_sigma_k(tbl_hbm, u_hbm, s_out, ssq_out, *refs):
        wv = [refs[:rpw], refs[rpw:2 * rpw]]
        u_v, sv, ssqv = refs[2 * rpw:2 * rpw + 3]
        sems = refs[2 * rpw + 3:]
        wid = lax.axis_index("s") * nc + lax.axis_index("c")
        row0 = base_row + wid * rpw

        pltpu.sync_copy(u_hbm, u_v)

        def chunk_copies(c0, buf):
            return [
                pltpu.make_async_copy(
                    tbl_hbm.at[pl.ds(row0 + r, 1), pl.ds(c0 * ch, ch)],
                    wv[buf][r], sems[buf])
                for r in range(rpw)
            ]

        accs = tuple(jnp.zeros((16,), jnp.float32) for _ in range(rpw))
        for cp in chunk_copies(0, 0):
            cp.start()
        for c0 in range(_SC_NCH):
            buf = c0 % 2
            if c0 + 1 < _SC_NCH:
                for cp in chunk_copies(c0 + 1, 1 - buf):
                    cp.start()
            for cp in chunk_copies(c0, buf):
                cp.wait()
            ubase = c0 * ch

            def dot_body(i, a):
                sl = pl.ds(i * 16, 16)
                uvec = u_v[0, pl.ds(ubase + i * 16, 16)]
                return tuple(a[r] + wv[buf][r][0, sl] * uvec
                             for r in range(rpw))
            accs = lax.fori_loop(0, nv, dot_body, accs)

        lane = lax.iota(jnp.int32, 16)

        dnums = lax.GatherDimensionNumbers(
            offset_dims=(), collapsed_slice_dims=(0,), start_index_map=(0,))

        def _perm(x, idx):
            return lax.gather(x, idx[:, None], dimension_numbers=dnums,
                              slice_sizes=(1,),
                              mode=lax.GatherScatterMode.PROMISE_IN_BOUNDS)

        def _allsum16(x):
            for sh in (1, 2, 4, 8):
                x = x + _perm(x, lane ^ sh)
            return x

        t1 = [_allsum16(a) for a in accs]
        ssq_vec = sum(t * t for t in t1)

        for cp in chunk_copies(0, 0):
            cp.start()
        for c0 in range(_SC_NCH):
            buf = c0 % 2
            if c0 + 1 < _SC_NCH:
                for cp in chunk_copies(c0 + 1, 1 - buf):
                    cp.start()
            for cp in chunk_copies(c0, buf):
                cp.wait()

            def s_body(i, _):
                sl = pl.ds(i * 16, 16)
                v = t1[0] * wv[buf][0][0, sl]
                for r in range(1, rpw):
                    v = v + t1[r] * wv[buf][r][0, sl]
                sv[0, sl] = v
                return 0
            lax.fori_loop(0, nv, s_body, 0)
            pltpu.sync_copy(
                sv, s_out.at[pl.ds(wid, 1), pl.ds(c0 * ch, ch)])

        ssqv[0, :] = jnp.where(lane == 0, ssq_vec, 0.0)
        pltpu.sync_copy(ssqv, ssq_out.at[pl.ds(wid, 1)])

    return sc_sigma_k


def _cond_body(s_tc_ref, ssq_tc_ref, s_sc_ref, ssq_sc_ref,
               t_ref, e_ref, o_ref, inv_ref):
    i = pl.program_id(0)

    @pl.when(i == 0)
    def _merge_sigma():
        s = s_tc_ref[...] + jnp.sum(s_sc_ref[...], axis=0, keepdims=True)
        s_sq = jnp.sum(s * s)
        ssq = ssq_tc_ref[0, 0] + jnp.sum(ssq_sc_ref[...])
        n1 = jnp.sqrt(ssq)
        d1 = n1 + _EPS
        t2_sq = s_sq / (d1 * d1)
        n2 = jnp.sqrt(t2_sq)
        inv_ref[0] = (n2 + _EPS) / t2_sq

    inv = inv_ref[0]
    n_w = t_ref.shape[1]
    c = t_ref.shape[3]
    for w_i in range(n_w):
        o_ref[0, w_i] = (t_ref[0, w_i]
                         + e_ref[:, pl.ds(w_i * c, c)] * inv)


def _cond_call(s_tc, ssq_tc, s_sc, ssq_sc, tensor_t, emb):
    h, w_, b, c = tensor_t.shape
    f = s_tc.shape[1]
    npart = s_sc.shape[0]
    return pl.pallas_call(
        _cond_body,
        grid=(h,),
        in_specs=[
            pl.BlockSpec((1, f), lambda i: (0, 0)),
            pl.BlockSpec(memory_space=pltpu.SMEM),
            pl.BlockSpec((npart, f), lambda i: (0, 0)),
            pl.BlockSpec((npart, 16), lambda i: (0, 0)),
            pl.BlockSpec((1, w_, b, c), lambda i: (i, 0, 0, 0)),
            pl.BlockSpec((b, w_ * c), lambda i: (0, i)),
        ],
        out_specs=pl.BlockSpec((1, w_, b, c), lambda i: (i, 0, 0, 0)),
        out_shape=jax.ShapeDtypeStruct((h, w_, b, c), jnp.float32),
        scratch_shapes=[
            pltpu.SMEM((1,), jnp.float32),
        ],
        compiler_params=pltpu.CompilerParams(
            dimension_semantics=("arbitrary",),
        ),
    )(s_tc, ssq_tc, s_sc, ssq_sc, tensor_t, emb)


def kernel(tensor, labels, table, u):
    b, h, w_, c = tensor.shape
    n_cls, f = table.shape

    n_tc = n_cls - _SC_ROWS
    s_tc, ssq_tc = _sigma_call(table, u, n_tc)
    s_sc, ssq_sc = _sc_sigma_fn(n_cls, f, n_tc)(table, u)

    labels32 = labels.astype(jnp.int32)
    emb = _sc_gather_fn(n_cls, f, b)(table, labels32)

    tensor_t = jnp.transpose(tensor, (1, 2, 0, 3))
    out_t = _cond_call(s_tc, ssq_tc, s_sc, ssq_sc, tensor_t, emb)
    return jnp.transpose(out_t, (2, 0, 1, 3))


# speedup vs baseline: 1.2368x; 1.2368x over previous
import functools

import jax
import jax.numpy as jnp
from jax import lax
from jax.experimental import pallas as pl
from jax.experimental.pallas import tpu as pltpu
from jax.experimental.pallas import tpu_sc as plsc

_EPS = 1e-12
_NCH = 12
_ROW_BLK = 40
_CW = 128
_NACC = 8
_CND_BLK = 8
_SB = 8


def _sigma_body(u_ref, w_ref, sig_ref, acc_ref, ssq_ref):
    i = pl.program_id(0)

    @pl.when(i == 0)
    def _init():
        acc_ref[...] = jnp.zeros_like(acc_ref)
        ssq_ref[0] = 0.0

    f = w_ref.shape[1]
    nchk = f // _CW
    accs = [None] * _NACC
    for k in range(nchk):
        sl = pl.ds(k * _CW, _CW)
        c = w_ref[:, sl] * u_ref[:, sl]
        j = k % _NACC
        accs[j] = c if accs[j] is None else accs[j] + c
    while len(accs) > 1:
        accs = [a + b for a, b in zip(accs[::2], accs[1::2])]
    t1 = jnp.sum(accs[0], axis=1, keepdims=True)
    ssq_ref[0] += jnp.sum(t1 * t1)
    t1b = jnp.broadcast_to(t1, (t1.shape[0], _CW))
    for k in range(nchk):
        sl = pl.ds(k * _CW, _CW)
        acc_ref[:, sl] += t1b * w_ref[:, sl]

    @pl.when(i == pl.num_programs(0) - 1)
    def _fin():
        s = jnp.sum(acc_ref[...], axis=0, keepdims=True)
        s_sq = jnp.sum(s * s)
        n1 = jnp.sqrt(ssq_ref[0])
        d1 = n1 + _EPS
        t2_sq = s_sq / (d1 * d1)
        n2 = jnp.sqrt(t2_sq)
        sig_ref[0, 0] = t2_sq / (n2 + _EPS)


def _sigma_call(table, u):
    n_cls, f = table.shape
    grid = n_cls // _ROW_BLK
    return pl.pallas_call(
        _sigma_body,
        grid=(grid,),
        in_specs=[
            pl.BlockSpec((1, f), lambda i: (0, 0)),
            pl.BlockSpec((_ROW_BLK, f), lambda i: (i, 0)),
        ],
        out_specs=pl.BlockSpec(memory_space=pltpu.SMEM),
        out_shape=jax.ShapeDtypeStruct((1, 1), jnp.float32),
        scratch_shapes=[
            pltpu.VMEM((_ROW_BLK, f), jnp.float32),
            pltpu.SMEM((1,), jnp.float32),
        ],
        compiler_params=pltpu.CompilerParams(
            dimension_semantics=("arbitrary",),
        ),
    )(u, table)


@functools.lru_cache(maxsize=None)
def _sc_gather_fn(n_cls, f, b):
    try:
        info = plsc.get_sparse_core_info()
        nc, ns = info.num_cores, info.num_subcores
    except Exception:
        nc, ns = 2, 16
    nw = nc * ns
    ch = f // _NCH
    ngrp = b // _SB
    nbat = ngrp * _NCH
    bpw = nbat // nw

    mesh = plsc.VectorSubcoreMesh(
        core_axis_name="c", subcore_axis_name="s",
        num_cores=nc, num_subcores=ns,
    )

    @functools.partial(
        pl.kernel,
        mesh=mesh,
        out_type=jax.ShapeDtypeStruct((b, f), jnp.float32),
        scratch_types=[
            pltpu.VMEM((_SB,), jnp.int32),
            pltpu.VMEM((_SB, ch), jnp.float32),
            pltpu.VMEM((_SB, ch), jnp.float32),
            pltpu.SemaphoreType.DMA,
            pltpu.SemaphoreType.DMA,
        ],
    )
    def gather_k(tbl_hbm, lbl_hbm, out_hbm, idx_v, b0, b1, s0, s1):
        wid = lax.axis_index("s") * nc + lax.axis_index("c")
        grp = wid % ngrp
        cb = (wid // ngrp) * bpw
        pltpu.sync_copy(lbl_hbm.at[pl.ds(grp * _SB, _SB)], idx_v)
        bufs = (b0, b1)
        sems = (s0, s1)
        copies = [
            pltpu.make_async_copy(
                tbl_hbm.at[idx_v, pl.ds((cb + k) * ch, ch)],
                bufs[k % 2],
                sems[k % 2],
            )
            for k in range(bpw)
        ]
        copies[0].start()
        for k in range(bpw):
            if k + 1 < bpw:
                copies[k + 1].start()
            copies[k].wait()
            pltpu.sync_copy(
                bufs[k % 2],
                out_hbm.at[pl.ds(grp * _SB, _SB),
                           pl.ds((cb + k) * ch, ch)])

    return gather_k


def _cond_body(sig_ref, t_ref, e_ref, o_ref):
    inv = 1.0 / sig_ref[0, 0]
    n_w = t_ref.shape[1]
    c = t_ref.shape[3]
    for w_i in range(n_w):
        o_ref[0, w_i] = (t_ref[0, w_i]
                         + e_ref[:, pl.ds(w_i * c, c)] * inv)


def _cond_call(sig, tensor_t, emb):
    h, w_, b, c = tensor_t.shape
    return pl.pallas_call(
        _cond_body,
        grid=(h,),
        in_specs=[
            pl.BlockSpec(memory_space=pltpu.SMEM),
            pl.BlockSpec((1, w_, b, c), lambda i: (i, 0, 0, 0)),
            pl.BlockSpec((b, w_ * c), lambda i: (0, i)),
        ],
        out_specs=pl.BlockSpec((1, w_, b, c), lambda i: (i, 0, 0, 0)),
        out_shape=jax.ShapeDtypeStruct((h, w_, b, c), jnp.float32),
        compiler_params=pltpu.CompilerParams(
            dimension_semantics=("parallel",),
        ),
    )(sig, tensor_t, emb)


def kernel(tensor, labels, table, u):
    b, h, w_, c = tensor.shape
    n_cls, f = table.shape

    sig = _sigma_call(table, u)

    labels32 = labels.astype(jnp.int32)
    emb = _sc_gather_fn(n_cls, f, b)(table, labels32)

    tensor_t = jnp.transpose(tensor, (1, 2, 0, 3))
    out_t = _cond_call(sig, tensor_t, emb)
    return jnp.transpose(out_t, (2, 0, 1, 3))


# speedup vs baseline: 1.2515x; 1.0118x over previous
import functools

import jax
import jax.numpy as jnp
from jax import lax
from jax.experimental import pallas as pl
from jax.experimental.pallas import tpu as pltpu
from jax.experimental.pallas import tpu_sc as plsc

_EPS = 1e-12
_NCH = 12
_ROW_BLK = 40
_CW = 128
_NACC = 8
_CND_BLK = 8
_SB = 8


def _sigma_body(u_ref, w_ref, sig_ref, acc_ref, ssq_ref):
    i = pl.program_id(0)

    @pl.when(i == 0)
    def _init():
        acc_ref[...] = jnp.zeros_like(acc_ref)
        ssq_ref[0] = 0.0

    f = w_ref.shape[1]
    nchk = f // _CW
    accs = [None] * _NACC
    for k in range(nchk):
        sl = pl.ds(k * _CW, _CW)
        c = w_ref[:, sl] * u_ref[:, sl]
        j = k % _NACC
        accs[j] = c if accs[j] is None else accs[j] + c
    while len(accs) > 1:
        accs = [a + b for a, b in zip(accs[::2], accs[1::2])]
    t1 = jnp.sum(accs[0], axis=1, keepdims=True)
    ssq_ref[0] += jnp.sum(t1 * t1)
    t1b = jnp.broadcast_to(t1, (t1.shape[0], _CW))
    for k in range(nchk):
        sl = pl.ds(k * _CW, _CW)
        acc_ref[:, sl] += t1b * w_ref[:, sl]

    @pl.when(i == pl.num_programs(0) - 1)
    def _fin():
        s = jnp.sum(acc_ref[...], axis=0, keepdims=True)
        s_sq = jnp.sum(s * s)
        n1 = jnp.sqrt(ssq_ref[0])
        d1 = n1 + _EPS
        t2_sq = s_sq / (d1 * d1)
        n2 = jnp.sqrt(t2_sq)
        sig_ref[0, 0] = t2_sq / (n2 + _EPS)


def _sigma_call(table, u):
    n_cls, f = table.shape
    grid = n_cls // _ROW_BLK
    return pl.pallas_call(
        _sigma_body,
        grid=(grid,),
        in_specs=[
            pl.BlockSpec((1, f), lambda i: (0, 0)),
            pl.BlockSpec((_ROW_BLK, f), lambda i: (i, 0)),
        ],
        out_specs=pl.BlockSpec(memory_space=pltpu.SMEM),
        out_shape=jax.ShapeDtypeStruct((1, 1), jnp.float32),
        scratch_shapes=[
            pltpu.VMEM((_ROW_BLK, f), jnp.float32),
            pltpu.SMEM((1,), jnp.float32),
        ],
        compiler_params=pltpu.CompilerParams(
            dimension_semantics=("arbitrary",),
        ),
    )(u, table)


@functools.lru_cache(maxsize=None)
def _sc_gather_fn(n_cls, f, b):
    try:
        info = plsc.get_sparse_core_info()
        nc, ns = info.num_cores, info.num_subcores
    except Exception:
        nc, ns = 2, 16
    nw = nc * ns
    ch = f // _NCH
    ngrp = b // _SB
    nbat = ngrp * _NCH
    bpw = nbat // nw

    mesh = plsc.VectorSubcoreMesh(
        core_axis_name="c", subcore_axis_name="s",
        num_cores=nc, num_subcores=ns,
    )

    @functools.partial(
        pl.kernel,
        mesh=mesh,
        out_type=jax.ShapeDtypeStruct((b, f), jnp.float32),
        scratch_types=[
            pltpu.VMEM((_SB,), jnp.int32),
            pltpu.VMEM((_SB, ch), jnp.float32),
            pltpu.VMEM((_SB, ch), jnp.float32),
            pltpu.SemaphoreType.DMA,
            pltpu.SemaphoreType.DMA,
        ],
    )
    def gather_k(tbl_hbm, lbl_hbm, out_hbm, idx_v, b0, b1, s0, s1):
        wid = lax.axis_index("s") * nc + lax.axis_index("c")
        grp = wid % ngrp
        cb = (wid // ngrp) * bpw
        pltpu.sync_copy(lbl_hbm.at[pl.ds(grp * _SB, _SB)], idx_v)
        bufs = (b0, b1)
        sems = (s0, s1)
        copies = [
            pltpu.make_async_copy(
                tbl_hbm.at[idx_v, pl.ds((cb + k) * ch, ch)],
                bufs[k % 2],
                sems[k % 2],
            )
            for k in range(bpw)
        ]
        copies[0].start()
        for k in range(bpw):
            if k + 1 < bpw:
                copies[k + 1].start()
            copies[k].wait()
            pltpu.sync_copy(
                bufs[k % 2],
                out_hbm.at[pl.ds(grp * _SB, _SB),
                           pl.ds((cb + k) * ch, ch)])

    return gather_k


def _cond_body(sig_ref, t_ref, e_ref, o_ref):
    inv = 1.0 / sig_ref[0, 0]
    g_blk, n_w, _, c = t_ref.shape
    for g in range(g_blk):
        for w_i in range(n_w):
            o_ref[g, w_i] = (t_ref[g, w_i]
                             + e_ref[:, pl.ds((g * n_w + w_i) * c, c)] * inv)


def _cond_call(sig, tensor_t, emb):
    h, w_, b, c = tensor_t.shape
    g_blk = 2
    return pl.pallas_call(
        _cond_body,
        grid=(h // g_blk,),
        in_specs=[
            pl.BlockSpec(memory_space=pltpu.SMEM),
            pl.BlockSpec((g_blk, w_, b, c), lambda i: (i, 0, 0, 0)),
            pl.BlockSpec((b, g_blk * w_ * c), lambda i: (0, i)),
        ],
        out_specs=pl.BlockSpec((g_blk, w_, b, c), lambda i: (i, 0, 0, 0)),
        out_shape=jax.ShapeDtypeStruct((h, w_, b, c), jnp.float32),
        compiler_params=pltpu.CompilerParams(
            dimension_semantics=("parallel",),
        ),
    )(sig, tensor_t, emb)


def kernel(tensor, labels, table, u):
    b, h, w_, c = tensor.shape
    n_cls, f = table.shape

    sig = _sigma_call(table, u)

    labels32 = labels.astype(jnp.int32)
    emb = _sc_gather_fn(n_cls, f, b)(table, labels32)

    tensor_t = jnp.transpose(tensor, (1, 2, 0, 3))
    out_t = _cond_call(sig, tensor_t, emb)
    return jnp.transpose(out_t, (2, 0, 1, 3))
